# baseline (speedup 1.0000x reference)
import functools

import jax
import jax.numpy as jnp
from jax import lax
from jax.experimental import pallas as pl
from jax.experimental.pallas import tpu as pltpu
from jax.experimental.pallas import tpu_sc as plsc

K = 1000000
OUT_F = 16384
IN_F = 64
B = OUT_F * IN_F

NC = 2
NS = 16
NW = NC * NS
BPW = B // NW
RPW = IN_F // NW

_MESH = plsc.VectorSubcoreMesh(
    core_axis_name="c", subcore_axis_name="s", num_cores=NC, num_subcores=NS
)


@functools.partial(
    pl.kernel,
    out_type=[
        jax.ShapeDtypeStruct((IN_F, OUT_F), jnp.float32),
        jax.ShapeDtypeStruct((IN_F, OUT_F), jnp.float32),
    ],
    mesh=_MESH,
    scratch_types=[
        pltpu.VMEM((BPW,), jnp.int32),
        pltpu.VMEM((OUT_F,), jnp.float32),
        pltpu.VMEM((OUT_F,), jnp.float32),
        pltpu.SemaphoreType.DMA,
        pltpu.SemaphoreType.DMA,
    ],
)
def _sc_gather(mu_hbm, rho_hbm, idx_hbm, mug_hbm, rhog_hbm,
               idx_v, mug_v, rhog_v, sem_mu, sem_rho):
    wid = lax.axis_index("s") * NC + lax.axis_index("c")
    base = wid * BPW
    pltpu.sync_copy(idx_hbm.at[pl.ds(base, BPW)], idx_v)
    for r in range(RPW):
        row_idx = idx_v.at[pl.ds(r * OUT_F, OUT_F)]
        cp_mu = pltpu.async_copy(mu_hbm.at[0].at[row_idx], mug_v, sem_mu)
        cp_rho = pltpu.async_copy(rho_hbm.at[0].at[row_idx], rhog_v, sem_rho)
        row = wid * RPW + r
        cp_mu.wait()
        pltpu.sync_copy(mug_v, mug_hbm.at[row, pl.ds(0, OUT_F)])
        cp_rho.wait()
        pltpu.sync_copy(rhog_v, rhog_hbm.at[row, pl.ds(0, OUT_F)])


_BLK = 16384


def _tc_finish_body(mu_ref, rho_ref, eps_ref, out_ref):
    sigma = jnp.log1p(jnp.exp(rho_ref[...]))
    out_ref[...] = mu_ref[...] + sigma * eps_ref[...]


_tc_finish = pl.pallas_call(
    _tc_finish_body,
    grid=(OUT_F // _BLK,),
    in_specs=[
        pl.BlockSpec((IN_F, _BLK), lambda i: (0, i)),
        pl.BlockSpec((IN_F, _BLK), lambda i: (0, i)),
        pl.BlockSpec((IN_F, _BLK), lambda i: (0, i)),
    ],
    out_specs=pl.BlockSpec((IN_F, _BLK), lambda i: (0, i)),
    out_shape=jax.ShapeDtypeStruct((IN_F, OUT_F), jnp.float32),
)


def kernel(weight_mu_share, weight_rho_share, eps_w, indices):
    idx_t = jnp.transpose(indices[0], (1, 0)).reshape(B)
    eps_t = jnp.transpose(eps_w[0], (1, 0))
    mu_g, rho_g = _sc_gather(weight_mu_share, weight_rho_share, idx_t)
    return _tc_finish(mu_g, rho_g, eps_t)


# speedup vs baseline: 1.0140x; 1.0140x over previous
import functools

import jax
import jax.numpy as jnp
from jax import lax
from jax.experimental import pallas as pl
from jax.experimental.pallas import tpu as pltpu
from jax.experimental.pallas import tpu_sc as plsc

K = 1000000
OUT_F = 16384
IN_F = 64
B = OUT_F * IN_F

NC = 2
NS = 16
NW = NC * NS
BPW = B // NW
RPW = IN_F // NW

_MESH = plsc.VectorSubcoreMesh(
    core_axis_name="c", subcore_axis_name="s", num_cores=NC, num_subcores=NS
)


@functools.partial(
    pl.kernel,
    out_type=[
        jax.ShapeDtypeStruct((IN_F, OUT_F), jnp.float32),
        jax.ShapeDtypeStruct((IN_F, OUT_F), jnp.float32),
    ],
    mesh=_MESH,
    scratch_types=[
        pltpu.VMEM((BPW,), jnp.int32),
        pltpu.VMEM((OUT_F,), jnp.float32),
        pltpu.VMEM((OUT_F,), jnp.float32),
        pltpu.SemaphoreType.DMA,
        pltpu.SemaphoreType.DMA,
    ],
)
def _sc_gather(mu_hbm, rho_hbm, idx_hbm, mug_hbm, rhog_hbm,
               idx_v, mug_v, rhog_v, sem_mu, sem_rho):
    wid = lax.axis_index("s") * NC + lax.axis_index("c")
    base = wid * BPW
    pltpu.sync_copy(idx_hbm.at[pl.ds(base, BPW)], idx_v)
    for r in range(RPW):
        row_idx = idx_v.at[pl.ds(r * OUT_F, OUT_F)]
        cp_mu = pltpu.async_copy(mu_hbm.at[0].at[row_idx], mug_v, sem_mu)
        cp_rho = pltpu.async_copy(rho_hbm.at[0].at[row_idx], rhog_v, sem_rho)
        row = wid * RPW + r
        cp_mu.wait()
        pltpu.sync_copy(mug_v, mug_hbm.at[row, pl.ds(0, OUT_F)])
        cp_rho.wait()
        pltpu.sync_copy(rhog_v, rhog_hbm.at[row, pl.ds(0, OUT_F)])


_BLK = 8192


def _tc_finish_body(mu_ref, rho_ref, eps_ref, out_ref):
    sigma = jnp.log1p(jnp.exp(rho_ref[...]))
    out_ref[...] = mu_ref[...] + sigma * eps_ref[...]


_tc_finish = pl.pallas_call(
    _tc_finish_body,
    grid=(OUT_F // _BLK,),
    in_specs=[
        pl.BlockSpec((IN_F, _BLK), lambda i: (0, i)),
        pl.BlockSpec((IN_F, _BLK), lambda i: (0, i)),
        pl.BlockSpec((IN_F, _BLK), lambda i: (0, i)),
    ],
    out_specs=pl.BlockSpec((IN_F, _BLK), lambda i: (0, i)),
    out_shape=jax.ShapeDtypeStruct((IN_F, OUT_F), jnp.float32),
)


def kernel(weight_mu_share, weight_rho_share, eps_w, indices):
    idx_t = jnp.transpose(indices[0], (1, 0)).reshape(B)
    eps_t = jnp.transpose(eps_w[0], (1, 0))
    mu_g, rho_g = _sc_gather(weight_mu_share, weight_rho_share, idx_t)
    return _tc_finish(mu_g, rho_g, eps_t)
